# baseline (speedup 1.0000x reference)
import jax
import jax.numpy as jnp
from jax import lax
from jax.experimental import pallas as pl
from jax.experimental.pallas import tpu as pltpu
from jax.experimental.pallas import tpu_sc as plsc

_NUM_CORES = 2

_ROWS = 16
_COLS = 1_000_000
_LAST = _COLS % 128

_NBUF = 3
_BLKC = 32_768
_CHUNK = 499_968


def _blocks(total, max_blk):
    plan, off = [], 0
    while off < total:
        w = min(max_blk, total - off)
        plan.append((off, w))
        off += w
    return plan


def _copy_body(table_hbm, out_hbm, bufs, tail_buf, rsems, wsems):
    cid = lax.axis_index("c")
    base = cid * _CHUNK

    def read(i, off, width):
        return pltpu.async_copy(
            table_hbm.at[:, pl.ds(base + off, width)],
            bufs[i % _NBUF].at[:, pl.ds(0, width)],
            rsems[i % _NBUF],
        )

    def write(i, off, width):
        return pltpu.async_copy(
            bufs[i % _NBUF].at[:, pl.ds(0, width)],
            out_hbm.at[:, pl.ds(base + off, width)],
            wsems[i % _NBUF],
        )

    plan = _blocks(_CHUNK, _BLKC)
    n = len(plan)
    rd = [None] * n
    wr = [None] * n
    for i in range(min(_NBUF, n)):
        rd[i] = read(i, *plan[i])
    for i in range(n):
        rd[i].wait()
        wr[i] = write(i, *plan[i])
        if i + _NBUF < n:
            wr[i].wait()
            rd[i + _NBUF] = read(i + _NBUF, *plan[i + _NBUF])
    for i in range(max(n - _NBUF, 0), n):
        wr[i].wait()

    if _LAST:
        off = _COLS - _LAST

        @pl.when(cid == _NUM_CORES - 1)
        def _():
            pltpu.sync_copy(table_hbm.at[:, pl.ds(off, _LAST)], tail_buf)
            pltpu.sync_copy(tail_buf, out_hbm.at[:, pl.ds(off, _LAST)])


def kernel(embed_node):
    xt = embed_node.T
    mesh = plsc.ScalarSubcoreMesh(axis_name="c", num_cores=_NUM_CORES)
    fn = pl.kernel(
        _copy_body,
        out_type=jax.ShapeDtypeStruct(xt.shape, xt.dtype),
        mesh=mesh,
        scratch_types=[
            [pltpu.VMEM_SHARED((_ROWS, _BLKC), jnp.float32) for _ in range(_NBUF)],
            pltpu.VMEM_SHARED((_ROWS, _LAST), jnp.float32),
            [pltpu.SemaphoreType.DMA for _ in range(_NBUF)],
            [pltpu.SemaphoreType.DMA for _ in range(_NBUF)],
        ],
        compiler_params=pltpu.CompilerParams(use_tc_tiling_on_sc=True),
    )
    return fn(xt).T
